# baseline (speedup 1.0000x reference)
import functools

import jax
import jax.numpy as jnp
from jax import lax
from jax.experimental import pallas as pl
from jax.experimental.pallas import tpu as pltpu
from jax.experimental.pallas import tpu_sc as plsc

N = 2048
D = 768
F = 2048
E = 8
T = 64
G = N // T + E
NPAD = G * T

_SC_CORES = 2
_SC_SUBCORES = 16
NW = _SC_CORES * _SC_SUBCORES
RW = N // NW


def _router_body(x_ref, wr_ref, dst_ref, toff_ref, tcnt_ref):
    x = x_ref[...]
    wr = wr_ref[...]
    logits = lax.dot_general(x, wr, (((1,), (1,)), ((), ())),
                             preferred_element_type=jnp.float32)
    m = jnp.max(logits, axis=1, keepdims=True)
    lane = lax.broadcasted_iota(jnp.int32, (N, E), 1)
    eid = jnp.min(jnp.where(logits == m, lane, E), axis=1, keepdims=True)
    onehot = (lane == eid).astype(jnp.int32)
    inc = onehot
    k = 1
    while k < N:
        inc = inc + jnp.concatenate(
            [jnp.zeros((k, E), jnp.int32), inc[:-k, :]], axis=0)
        k *= 2
    counts = inc[-1:, :]
    pc = ((counts + (T - 1)) // T) * T
    seg_end = pc
    k = 1
    while k < E:
        seg_end = seg_end + jnp.concatenate(
            [jnp.zeros((1, k), jnp.int32), seg_end[:, :-k]], axis=1)
        k *= 2
    seg_start = seg_end - pc
    dst_ref[...] = jnp.sum(onehot * (seg_start + inc - 1),
                           axis=1, keepdims=True)
    counts_col = lax.dot_general(
        onehot.astype(jnp.float32), jnp.ones((N, 1), jnp.float32),
        (((0,), (0,)), ((), ())),
        preferred_element_type=jnp.float32).astype(jnp.int32)
    pc_col = ((counts_col + (T - 1)) // T) * T
    end_col = pc_col
    k = 1
    while k < E:
        end_col = end_col + jnp.concatenate(
            [jnp.zeros((k, 1), jnp.int32), end_col[:-k, :]], axis=0)
        k *= 2
    toff_ref[...] = (end_col - pc_col) // T
    tcnt_ref[...] = pc_col // T


_router_call = pl.pallas_call(
    _router_body,
    out_shape=(
        jax.ShapeDtypeStruct((N, 1), jnp.int32),
        jax.ShapeDtypeStruct((E, 1), jnp.int32),
        jax.ShapeDtypeStruct((E, 1), jnp.int32),
    ),
)


_NF = 1
_FB = F // _NF


def _mlp_body(toff_ref, tcnt_ref, x_ref, wg_ref, wu_ref, wd_ref, y_ref):
    e = pl.program_id(0)
    f = pl.program_id(1)
    t0 = toff_ref[e]
    nt = tcnt_ref[e]
    dn = (((1,), (0,)), ((), ()))

    def tile_body(j, carry):
        rows = pl.ds((t0 + j) * T, T)
        xb = x_ref[rows, :]
        g = lax.dot_general(xb, wg_ref[0], dn, preferred_element_type=jnp.float32)
        u = lax.dot_general(xb, wu_ref[0], dn, preferred_element_type=jnp.float32)
        part = lax.dot_general((g * jax.nn.sigmoid(g) * u), wd_ref[0], dn,
                               preferred_element_type=jnp.float32)

        @pl.when(f == 0)
        def _():
            y_ref[rows, :] = part

        @pl.when(f != 0)
        def _():
            y_ref[rows, :] += part

        return carry

    lax.fori_loop(0, nt, tile_body, 0)


_mlp_call = pl.pallas_call(
    _mlp_body,
    grid_spec=pltpu.PrefetchScalarGridSpec(
        num_scalar_prefetch=2,
        grid=(E, _NF),
        in_specs=[
            pl.BlockSpec((NPAD, D), lambda e, f, toff, tcnt: (0, 0)),
            pl.BlockSpec((1, D, _FB), lambda e, f, toff, tcnt: (e, 0, f)),
            pl.BlockSpec((1, D, _FB), lambda e, f, toff, tcnt: (e, 0, f)),
            pl.BlockSpec((1, _FB, D), lambda e, f, toff, tcnt: (e, f, 0)),
        ],
        out_specs=pl.BlockSpec((NPAD, D), lambda e, f, toff, tcnt: (0, 0)),
    ),
    out_shape=jax.ShapeDtypeStruct((NPAD, D), jnp.float32),
)

_sc_mesh = plsc.VectorSubcoreMesh(core_axis_name="c", subcore_axis_name="s")


@functools.partial(
    pl.kernel,
    out_type=jax.ShapeDtypeStruct((NPAD, D), jnp.float32),
    mesh=_sc_mesh,
    scratch_types=[
        pltpu.VMEM((RW,), jnp.int32),
        pltpu.VMEM((RW, D), jnp.float32),
        pltpu.SemaphoreType.DMA,
    ],
)
def _sc_scatter_rows(x_hbm, dst_hbm, out_hbm, idx_v, rows_v, sem):
    wid = lax.axis_index("s") * _SC_CORES + lax.axis_index("c")
    base = wid * RW
    pltpu.sync_copy(dst_hbm.at[pl.ds(base, RW)], idx_v)
    pltpu.sync_copy(x_hbm.at[pl.ds(base, RW)], rows_v)
    pltpu.async_copy(rows_v, out_hbm.at[idx_v], sem).wait()


@functools.partial(
    pl.kernel,
    out_type=jax.ShapeDtypeStruct((N, D), jnp.float32),
    mesh=_sc_mesh,
    scratch_types=[
        pltpu.VMEM((RW,), jnp.int32),
        pltpu.VMEM((RW, D), jnp.float32),
        pltpu.SemaphoreType.DMA,
    ],
)
def _sc_gather_rows(y_hbm, dst_hbm, out_hbm, idx_v, rows_v, sem):
    wid = lax.axis_index("s") * _SC_CORES + lax.axis_index("c")
    base = wid * RW
    pltpu.sync_copy(dst_hbm.at[pl.ds(base, RW)], idx_v)
    pltpu.async_copy(y_hbm.at[idx_v], rows_v, sem).wait()
    pltpu.sync_copy(rows_v, out_hbm.at[pl.ds(base, RW)])


def kernel(hidden_states, Wr, Wg, Wu, Wd):
    s, b, d = hidden_states.shape
    x = hidden_states.reshape(N, D)
    dst2, toff2, tcnt2 = _router_call(x, Wr)
    dst = dst2.reshape(N)
    toff = toff2.reshape(E)
    tcnt = tcnt2.reshape(E)
    x_sorted = _sc_scatter_rows(x, dst)
    y_sorted = _mlp_call(toff, tcnt, x_sorted, Wg, Wu, Wd)
    out = _sc_gather_rows(y_sorted, dst)
    return out.reshape(s, b, d)


# speedup vs baseline: 1.0117x; 1.0117x over previous
import functools

import jax
import jax.numpy as jnp
from jax import lax
from jax.experimental import pallas as pl
from jax.experimental.pallas import tpu as pltpu
from jax.experimental.pallas import tpu_sc as plsc

N = 2048
D = 768
F = 2048
E = 8
T = 128
G = N // T + E
NPAD = G * T

_SC_CORES = 2
_SC_SUBCORES = 16
NW = _SC_CORES * _SC_SUBCORES
RW = N // NW


def _router_body(x_ref, wr_ref, dst_ref, toff_ref, tcnt_ref):
    x = x_ref[...]
    wr = wr_ref[...]
    logits = lax.dot_general(x, wr, (((1,), (1,)), ((), ())),
                             preferred_element_type=jnp.float32)
    m = jnp.max(logits, axis=1, keepdims=True)
    lane = lax.broadcasted_iota(jnp.int32, (N, E), 1)
    eid = jnp.min(jnp.where(logits == m, lane, E), axis=1, keepdims=True)
    onehot = (lane == eid).astype(jnp.int32)
    inc = onehot
    k = 1
    while k < N:
        inc = inc + jnp.concatenate(
            [jnp.zeros((k, E), jnp.int32), inc[:-k, :]], axis=0)
        k *= 2
    counts = inc[-1:, :]
    pc = ((counts + (T - 1)) // T) * T
    seg_end = pc
    k = 1
    while k < E:
        seg_end = seg_end + jnp.concatenate(
            [jnp.zeros((1, k), jnp.int32), seg_end[:, :-k]], axis=1)
        k *= 2
    seg_start = seg_end - pc
    dst_ref[...] = jnp.sum(onehot * (seg_start + inc - 1),
                           axis=1, keepdims=True)
    counts_col = lax.dot_general(
        onehot.astype(jnp.float32), jnp.ones((N, 1), jnp.float32),
        (((0,), (0,)), ((), ())),
        preferred_element_type=jnp.float32).astype(jnp.int32)
    pc_col = ((counts_col + (T - 1)) // T) * T
    end_col = pc_col
    k = 1
    while k < E:
        end_col = end_col + jnp.concatenate(
            [jnp.zeros((k, 1), jnp.int32), end_col[:-k, :]], axis=0)
        k *= 2
    toff_ref[...] = (end_col - pc_col) // T
    tcnt_ref[...] = pc_col // T


_router_call = pl.pallas_call(
    _router_body,
    out_shape=(
        jax.ShapeDtypeStruct((N, 1), jnp.int32),
        jax.ShapeDtypeStruct((E, 1), jnp.int32),
        jax.ShapeDtypeStruct((E, 1), jnp.int32),
    ),
)


_NF = 4
_FB = F // _NF


def _mlp_body(toff_ref, tcnt_ref, x_ref, wg_ref, wu_ref, wd_ref, y_ref):
    e = pl.program_id(0)
    f = pl.program_id(1)
    t0 = toff_ref[e]
    nt = tcnt_ref[e]
    dn = (((1,), (0,)), ((), ()))

    def tile_body(j, carry):
        rows = pl.ds((t0 + j) * T, T)
        xb = x_ref[rows, :]
        g = lax.dot_general(xb, wg_ref[0], dn, preferred_element_type=jnp.float32)
        u = lax.dot_general(xb, wu_ref[0], dn, preferred_element_type=jnp.float32)
        part = lax.dot_general((g * jax.nn.sigmoid(g) * u), wd_ref[0], dn,
                               preferred_element_type=jnp.float32)

        @pl.when(f == 0)
        def _():
            y_ref[rows, :] = part

        @pl.when(f != 0)
        def _():
            y_ref[rows, :] += part

        return carry

    lax.fori_loop(0, nt, tile_body, 0)


_mlp_call = pl.pallas_call(
    _mlp_body,
    grid_spec=pltpu.PrefetchScalarGridSpec(
        num_scalar_prefetch=2,
        grid=(E, _NF),
        in_specs=[
            pl.BlockSpec((NPAD, D), lambda e, f, toff, tcnt: (0, 0)),
            pl.BlockSpec((1, D, _FB), lambda e, f, toff, tcnt: (e, 0, f)),
            pl.BlockSpec((1, D, _FB), lambda e, f, toff, tcnt: (e, 0, f)),
            pl.BlockSpec((1, _FB, D), lambda e, f, toff, tcnt: (e, f, 0)),
        ],
        out_specs=pl.BlockSpec((NPAD, D), lambda e, f, toff, tcnt: (0, 0)),
    ),
    out_shape=jax.ShapeDtypeStruct((NPAD, D), jnp.float32),
)

_sc_mesh = plsc.VectorSubcoreMesh(core_axis_name="c", subcore_axis_name="s")


@functools.partial(
    pl.kernel,
    out_type=jax.ShapeDtypeStruct((NPAD, D), jnp.float32),
    mesh=_sc_mesh,
    scratch_types=[
        pltpu.VMEM((RW,), jnp.int32),
        pltpu.VMEM((RW, D), jnp.float32),
        pltpu.SemaphoreType.DMA,
    ],
)
def _sc_scatter_rows(x_hbm, dst_hbm, out_hbm, idx_v, rows_v, sem):
    wid = lax.axis_index("s") * _SC_CORES + lax.axis_index("c")
    base = wid * RW
    pltpu.sync_copy(dst_hbm.at[pl.ds(base, RW)], idx_v)
    pltpu.sync_copy(x_hbm.at[pl.ds(base, RW)], rows_v)
    pltpu.async_copy(rows_v, out_hbm.at[idx_v], sem).wait()


@functools.partial(
    pl.kernel,
    out_type=jax.ShapeDtypeStruct((N, D), jnp.float32),
    mesh=_sc_mesh,
    scratch_types=[
        pltpu.VMEM((RW,), jnp.int32),
        pltpu.VMEM((RW, D), jnp.float32),
        pltpu.SemaphoreType.DMA,
    ],
)
def _sc_gather_rows(y_hbm, dst_hbm, out_hbm, idx_v, rows_v, sem):
    wid = lax.axis_index("s") * _SC_CORES + lax.axis_index("c")
    base = wid * RW
    pltpu.sync_copy(dst_hbm.at[pl.ds(base, RW)], idx_v)
    pltpu.async_copy(y_hbm.at[idx_v], rows_v, sem).wait()
    pltpu.sync_copy(rows_v, out_hbm.at[pl.ds(base, RW)])


def kernel(hidden_states, Wr, Wg, Wu, Wd):
    s, b, d = hidden_states.shape
    x = hidden_states.reshape(N, D)
    dst2, toff2, tcnt2 = _router_call(x, Wr)
    dst = dst2.reshape(N)
    toff = toff2.reshape(E)
    tcnt = tcnt2.reshape(E)
    x_sorted = _sc_scatter_rows(x, dst)
    y_sorted = _mlp_call(toff, tcnt, x_sorted, Wg, Wu, Wd)
    out = _sc_gather_rows(y_sorted, dst)
    return out.reshape(s, b, d)


# speedup vs baseline: 1.2124x; 1.1984x over previous
import functools

import jax
import jax.numpy as jnp
from jax import lax
from jax.experimental import pallas as pl
from jax.experimental.pallas import tpu as pltpu
from jax.experimental.pallas import tpu_sc as plsc

N = 2048
D = 768
F = 2048
E = 8
T = 128
G = N // T + E
NPAD = G * T

RPR = 6
N8 = N // 8
XREC = N * RPR
SREC = NPAD * RPR

_SC_CORES = 2
_SC_SUBCORES = 16
NW = _SC_CORES * _SC_SUBCORES
RW = N // NW
RW6 = RW * RPR


def _router_body(x_ref, wr_ref, xrec_ref, i6s_ref, i6g_ref, toff_ref,
                 tcnt_ref):
    x = x_ref[...]
    wr = wr_ref[...]
    logits = lax.dot_general(x, wr, (((1,), (1,)), ((), ())),
                             preferred_element_type=jnp.float32)
    m = jnp.max(logits, axis=1, keepdims=True)
    lane = lax.broadcasted_iota(jnp.int32, (N, E), 1)
    eid = jnp.min(jnp.where(logits == m, lane, E), axis=1, keepdims=True)
    onehot = (lane == eid).astype(jnp.int32)
    inc = onehot
    k = 1
    while k < N:
        inc = inc + jnp.concatenate(
            [jnp.zeros((k, E), jnp.int32), inc[:-k, :]], axis=0)
        k *= 2
    counts = inc[-1:, :]
    pc = ((counts + (T - 1)) // T) * T
    seg_end = pc
    k = 1
    while k < E:
        seg_end = seg_end + jnp.concatenate(
            [jnp.zeros((1, k), jnp.int32), seg_end[:, :-k]], axis=1)
        k *= 2
    seg_start = seg_end - pc
    dst = jnp.sum(onehot * (seg_start + inc - 1),
                  axis=1, keepdims=True)
    a = (dst // 8) * 48 + (dst % 8)
    i6g_ref[...] = a + 8 * lax.broadcasted_iota(jnp.int32, (N, RPR), 1)
    a8 = a.reshape(N8, 8)
    i6s_ref[...] = jnp.tile(a8, (1, RPR)) + 8 * (
        lax.broadcasted_iota(jnp.int32, (N8, 8 * RPR), 1) // 8)
    def cp(g, carry):
        for c in range(RPR):
            xrec_ref[pl.ds((g * RPR + c) * 8, 8), :] = (
                x_ref[pl.ds(g * 8, 8), pl.ds(c * 128, 128)])
        return carry
    lax.fori_loop(0, N8, cp, 0)
    counts_col = lax.dot_general(
        onehot.astype(jnp.float32), jnp.ones((N, 1), jnp.float32),
        (((0,), (0,)), ((), ())),
        preferred_element_type=jnp.float32).astype(jnp.int32)
    pc_col = ((counts_col + (T - 1)) // T) * T
    end_col = pc_col
    k = 1
    while k < E:
        end_col = end_col + jnp.concatenate(
            [jnp.zeros((k, 1), jnp.int32), end_col[:-k, :]], axis=0)
        k *= 2
    toff_ref[...] = (end_col - pc_col) // T
    tcnt_ref[...] = pc_col // T


_router_call = pl.pallas_call(
    _router_body,
    out_shape=(
        jax.ShapeDtypeStruct((XREC, 128), jnp.float32),
        jax.ShapeDtypeStruct((N8, 8 * RPR), jnp.int32),
        jax.ShapeDtypeStruct((N, RPR), jnp.int32),
        jax.ShapeDtypeStruct((E, 1), jnp.int32),
        jax.ShapeDtypeStruct((E, 1), jnp.int32),
    ),
)


def _mlp_body(toff_ref, tcnt_ref, x_ref, wg_ref, wu_ref, wd_ref, y_ref):
    e = pl.program_id(0)
    t0 = toff_ref[e]
    nt = tcnt_ref[e]
    dn = (((1,), (0,)), ((), ()))

    def tile_body(j, carry):
        b6 = (t0 + j) * (T * RPR)
        xb = jnp.concatenate([
            jnp.concatenate(
                [x_ref[pl.ds(b6 + g * 48 + c * 8, 8), :]
                 for c in range(RPR)], axis=1)
            for g in range(T // 8)], axis=0)
        g_ = lax.dot_general(xb, wg_ref[0], dn,
                             preferred_element_type=jnp.float32)
        u = lax.dot_general(xb, wu_ref[0], dn,
                            preferred_element_type=jnp.float32)
        part = lax.dot_general((g_ * jax.nn.sigmoid(g_) * u), wd_ref[0], dn,
                               preferred_element_type=jnp.float32)
        for g in range(T // 8):
            for c in range(RPR):
                y_ref[pl.ds(b6 + g * 48 + c * 8, 8), :] = (
                    part[g * 8:(g + 1) * 8, c * 128:(c + 1) * 128])
        return carry

    lax.fori_loop(0, nt, tile_body, 0)


_mlp_call = pl.pallas_call(
    _mlp_body,
    grid_spec=pltpu.PrefetchScalarGridSpec(
        num_scalar_prefetch=2,
        grid=(E,),
        in_specs=[
            pl.BlockSpec((SREC, 128), lambda e, toff, tcnt: (0, 0)),
            pl.BlockSpec((1, D, F), lambda e, toff, tcnt: (e, 0, 0)),
            pl.BlockSpec((1, D, F), lambda e, toff, tcnt: (e, 0, 0)),
            pl.BlockSpec((1, F, D), lambda e, toff, tcnt: (e, 0, 0)),
        ],
        out_specs=pl.BlockSpec((SREC, 128), lambda e, toff, tcnt: (0, 0)),
    ),
    out_shape=jax.ShapeDtypeStruct((SREC, 128), jnp.float32),
)

_sc_mesh = plsc.VectorSubcoreMesh(core_axis_name="c", subcore_axis_name="s")


@functools.partial(
    pl.kernel,
    out_type=jax.ShapeDtypeStruct((SREC, 128), jnp.float32),
    mesh=_sc_mesh,
    scratch_types=[
        pltpu.VMEM((RW6,), jnp.int32),
        pltpu.VMEM((RW6, 128), jnp.float32),
        pltpu.SemaphoreType.DMA,
    ],
)
def _sc_scatter_recs(xrec_hbm, i6s_hbm, out_hbm, idx_v, rows_v, sem):
    wid = lax.axis_index("s") * _SC_CORES + lax.axis_index("c")
    b = wid * RW6
    pltpu.sync_copy(i6s_hbm.at[pl.ds(b, RW6)], idx_v)
    pltpu.sync_copy(xrec_hbm.at[pl.ds(b, RW6)], rows_v)
    pltpu.async_copy(rows_v, out_hbm.at[idx_v], sem).wait()


@functools.partial(
    pl.kernel,
    out_type=jax.ShapeDtypeStruct((XREC, 128), jnp.float32),
    mesh=_sc_mesh,
    scratch_types=[
        pltpu.VMEM((RW6,), jnp.int32),
        pltpu.VMEM((RW6, 128), jnp.float32),
        pltpu.SemaphoreType.DMA,
    ],
)
def _sc_gather_recs(yrec_hbm, i6g_hbm, out_hbm, idx_v, rows_v, sem):
    wid = lax.axis_index("s") * _SC_CORES + lax.axis_index("c")
    b = wid * RW6
    pltpu.sync_copy(i6g_hbm.at[pl.ds(b, RW6)], idx_v)
    pltpu.async_copy(yrec_hbm.at[idx_v], rows_v, sem).wait()
    pltpu.sync_copy(rows_v, out_hbm.at[pl.ds(b, RW6)])


def kernel(hidden_states, Wr, Wg, Wu, Wd):
    s, b, d = hidden_states.shape
    x = hidden_states.reshape(N, D)
    xrec, i6s, i6g, toff2, tcnt2 = _router_call(x, Wr)
    i6s_f = i6s.reshape(XREC)
    i6g_f = i6g.reshape(XREC)
    toff = toff2.reshape(E)
    tcnt = tcnt2.reshape(E)
    xs = _sc_scatter_recs(xrec, i6s_f)
    ys = _mlp_call(toff, tcnt, xs, Wg, Wu, Wd)
    out = _sc_gather_recs(ys, i6g_f)
    return out.reshape(s, b, d)
